# baseline (speedup 1.0000x reference)
import jax
import jax.numpy as jnp
from jax.experimental import pallas as pl
from jax.experimental.pallas import tpu as pltpu

_ROWS, _COLS = 50176, 768
_BLK = 3136


def _copy_body(x_ref, o_ref):
    o_ref[...] = x_ref[...]


def kernel(x, center):
    del center
    flat = x.transpose(2, 3, 0, 1).reshape(_ROWS, _COLS)
    yt = pl.pallas_call(
        _copy_body,
        grid=(_ROWS // _BLK,),
        in_specs=[pl.BlockSpec((_BLK, _COLS), lambda i: (i, 0))],
        out_specs=pl.BlockSpec((_BLK, _COLS), lambda i: (i, 0)),
        out_shape=jax.ShapeDtypeStruct((_ROWS, _COLS), x.dtype),
        compiler_params=pltpu.CompilerParams(
            dimension_semantics=("arbitrary",),
        ),
    )(flat)
    return yt.reshape(14, 14, 256, 768).transpose(2, 3, 0, 1)


# speedup vs baseline: 1.0025x; 1.0025x over previous
import jax
import jax.numpy as jnp
from jax.experimental import pallas as pl
from jax.experimental.pallas import tpu as pltpu

_ROWS, _COLS = 50176, 768
_BLK = 3584


def _copy_body(x_ref, o_ref):
    o_ref[...] = x_ref[...]


def kernel(x, center):
    del center
    flat = x.transpose(2, 3, 0, 1).reshape(_ROWS, _COLS)
    yt = pl.pallas_call(
        _copy_body,
        grid=(_ROWS // _BLK,),
        in_specs=[pl.BlockSpec((_BLK, _COLS), lambda i: (i, 0))],
        out_specs=pl.BlockSpec((_BLK, _COLS), lambda i: (i, 0)),
        out_shape=jax.ShapeDtypeStruct((_ROWS, _COLS), x.dtype),
        compiler_params=pltpu.CompilerParams(
            dimension_semantics=("arbitrary",),
        ),
    )(flat)
    return yt.reshape(14, 14, 256, 768).transpose(2, 3, 0, 1)
